# baseline (speedup 1.0000x reference)
import functools
import math

import numpy as np
import jax
import jax.numpy as jnp
from jax import lax
from jax.experimental import pallas as pl
from jax.experimental.pallas import tpu as pltpu
from jax.experimental.pallas import tpu_sc as plsc

D_MODEL = 256
N_SAMPLES = 64
L = 16
NCORES = 2
NSUB = 16
WPB = 8
UNROLL = 8
_INF = np.float32(float("inf"))
_NINF = np.float32(-float("inf"))


def _make_pe_tiled(batch):
    pos = np.arange(N_SAMPLES)[:, None].astype(np.float32)
    div = np.exp(np.arange(0, D_MODEL, 2).astype(np.float32)
                 * (-math.log(10000.0) / D_MODEL))
    pe = np.zeros((N_SAMPLES, D_MODEL), dtype=np.float32)
    pe[:, 0::2] = np.sin(pos * div)
    pe[:, 1::2] = np.cos(pos * div)
    return np.tile(pe, (batch, 1))


def _build_rec(iota, mv, xb, yb, zb, wb):
    return jnp.where(iota == 0, mv,
           jnp.where(iota == 1, xb,
           jnp.where(iota == 2, yb,
           jnp.where(iota == 3, zb, wb))))


@functools.lru_cache(maxsize=None)
def _make_fps_kernel(batch, n_points, chunk):
    n_slices = chunk // (L * UNROLL)
    bpc = batch // NCORES

    mesh = plsc.VectorSubcoreMesh(core_axis_name="c", subcore_axis_name="s")

    @functools.partial(
        pl.kernel,
        out_type=jax.ShapeDtypeStruct((batch * N_SAMPLES * 4,), jnp.float32),
        mesh=mesh,
        compiler_params=pltpu.CompilerParams(needs_layout_passes=False),
        scratch_types=[
            pltpu.VMEM((chunk * 4,), jnp.float32),
            pltpu.VMEM((chunk,), jnp.float32),
            pltpu.VMEM((chunk,), jnp.float32),
            pltpu.VMEM((chunk,), jnp.float32),
            pltpu.VMEM((chunk,), jnp.float32),
            pltpu.VMEM((L,), jnp.float32),
            pltpu.VMEM((NSUB * L,), jnp.float32),
            pltpu.VMEM((N_SAMPLES * 4,), jnp.float32),
            pltpu.VMEM_SHARED((NSUB * L,), jnp.float32),
        ],
    )
    def fps_kernel(pts_hbm, out_hbm, raw_ref, x_ref, y_ref, z_ref, dist_ref,
                   rec_ref, recs_ref, samp_ref, shared_ref):
        c = lax.axis_index("c")
        s = lax.axis_index("s")
        b = c * bpc + s // WPB
        g = s % WPB
        b32 = jnp.minimum(g * chunk, n_points - chunk) // 32
        base = b32 * 32
        iota = lax.iota(jnp.int32, L)
        gb = (s // WPB) * WPB
        in_group = (iota >= gb) & (iota < gb + WPB)
        ch_mask = iota < 4

        row0 = b * (n_points * 4 // 128) + b32
        pltpu.sync_copy(pts_hbm.at[pl.ds(row0 * 128, chunk * 4)], raw_ref)
        inf_v = jnp.full((L,), _INF)
        i4 = iota * 4

        def deint_step(t, _):
            off = t * L
            idx = i4 + off * 4
            x_ref[pl.ds(off, L)] = plsc.load_gather(raw_ref, [idx])
            y_ref[pl.ds(off, L)] = plsc.load_gather(raw_ref, [idx + 1])
            z_ref[pl.ds(off, L)] = plsc.load_gather(raw_ref, [idx + 2])
            dist_ref[pl.ds(off, L)] = inf_v
            return 0
        lax.fori_loop(0, chunk // L, deint_step, 0, unroll=4)

        def publish_select(rec, i):
            rec_ref[...] = rec
            pltpu.sync_copy(rec_ref, shared_ref.at[pl.ds(s * L, L)])
            plsc.subcore_barrier()
            pltpu.sync_copy(shared_ref, recs_ref)
            plsc.subcore_barrier()
            m_all = plsc.load_gather(recs_ref, [iota * L])
            mval = jnp.where(in_group, m_all, _NINF)
            mmax = jnp.max(mval)
            wg = jnp.min(jnp.where(mval == mmax, iota, jnp.int32(64)))
            wb = wg * L
            wbv = jnp.full((L,), wb, jnp.int32)
            cx = plsc.load_gather(recs_ref, [wbv + 1])
            cy = plsc.load_gather(recs_ref, [wbv + 2])
            cz = plsc.load_gather(recs_ref, [wbv + 3])

            @pl.when(g == 0)
            def _():
                vals = plsc.load_gather(recs_ref, [wbv + 1 + iota],
                                        mask=ch_mask)
                plsc.store_scatter(samp_ref, [i * 4 + iota], vals,
                                   mask=ch_mask)

            return cx, cy, cz

        def _lane_splat(v, k):
            return jnp.full((L,), jnp.sum(jnp.where(iota == k, v, 0.0)))

        head = raw_ref[pl.ds(0, L)]
        x0 = _lane_splat(head, 0)
        y0 = _lane_splat(head, 1)
        z0 = _lane_splat(head, 2)
        w0 = _lane_splat(head, 3)
        m0 = jnp.where(g == 0, _INF, _NINF)
        rec0 = _build_rec(iota, jnp.full((L,), m0), x0, y0, z0, w0)
        carry0 = publish_select(rec0, jnp.int32(0))

        def fps_step(i, carry):
            cx, cy, cz = carry

            def upd(t, bc):
                bv, bp = bc
                off = t * L
                dx = x_ref[pl.ds(off, L)] - cx
                dy = y_ref[pl.ds(off, L)] - cy
                dz = z_ref[pl.ds(off, L)] - cz
                d2 = dx * dx + dy * dy + dz * dz
                nd = jnp.minimum(dist_ref[pl.ds(off, L)], d2)
                dist_ref[pl.ds(off, L)] = nd
                gt = nd > bv
                bv = jnp.where(gt, nd, bv)
                bp = jnp.where(gt, off + iota, bp)
                return bv, bp

            bv, bp = plsc.parallel_loop(
                0, chunk // L, unroll=UNROLL,
                carry=(jnp.full((L,), _NINF), iota))(upd)
            ml = jnp.max(bv)
            lp = jnp.min(jnp.where(bv == ml, bp, jnp.int32(1 << 30)))
            pv4 = jnp.full((L,), lp * 4, jnp.int32)
            xb = plsc.load_gather(raw_ref, [pv4])
            yb = plsc.load_gather(raw_ref, [pv4 + 1])
            zb = plsc.load_gather(raw_ref, [pv4 + 2])
            wb = plsc.load_gather(raw_ref, [pv4 + 3])
            rec = _build_rec(iota, jnp.full((L,), ml), xb, yb, zb, wb)
            return publish_select(rec, i + 1)

        lax.fori_loop(0, N_SAMPLES - 1, fps_step, carry0)

        @pl.when(g == 0)
        def _():
            pltpu.sync_copy(samp_ref,
                            out_hbm.at[pl.ds(b * (N_SAMPLES * 4),
                                             N_SAMPLES * 4)])

    return fps_kernel


def _mlp_body(s_ref, w1_ref, b1_ref, w2_ref, b2_ref, w3_ref, b3_ref,
              pe_ref, o_ref):
    h = jnp.dot(s_ref[...], w1_ref[...],
                preferred_element_type=jnp.float32) + b1_ref[...]
    h = jnp.maximum(h, 0.0)
    h = jnp.dot(h, w2_ref[...],
                preferred_element_type=jnp.float32) + b2_ref[...]
    h = jnp.maximum(h, 0.0)
    o_ref[...] = (jnp.dot(h, w3_ref[...],
                          preferred_element_type=jnp.float32)
                  + b3_ref[...] + pe_ref[...])


def kernel(points, W1, b1, W2, b2, W3, b3):
    batch, n_points, _ = points.shape
    m = min(N_SAMPLES, n_points)
    gran = L * UNROLL
    chunk = ((n_points + WPB - 1) // WPB + gran - 1) // gran * gran

    pts_flat = points.reshape(-1)
    samp = _make_fps_kernel(batch, n_points, chunk)(pts_flat)
    s2d = samp.reshape(batch * N_SAMPLES, 4)

    pe_tiled = jnp.asarray(_make_pe_tiled(batch))
    out = pl.pallas_call(
        _mlp_body,
        out_shape=jax.ShapeDtypeStruct((batch * N_SAMPLES, D_MODEL),
                                       jnp.float32),
    )(s2d, W1.T, b1.reshape(1, -1), W2.T, b2.reshape(1, -1),
      W3.T, b3.reshape(1, -1), pe_tiled)
    return out.reshape(batch, m, D_MODEL)


# speedup vs baseline: 4.2726x; 4.2726x over previous
import functools
import math

import numpy as np
import jax
import jax.numpy as jnp
from jax import lax
from jax.experimental import pallas as pl
from jax.experimental.pallas import tpu as pltpu
from jax.experimental.pallas import tpu_sc as plsc

D_MODEL = 256
N_SAMPLES = 64
L = 16
NCORES = 2
NSUB = 16
WPB = 8
UNROLL = 8
_INF = np.float32(float("inf"))
_NINF = np.float32(-float("inf"))


def _make_pe_tiled(batch):
    pos = np.arange(N_SAMPLES)[:, None].astype(np.float32)
    div = np.exp(np.arange(0, D_MODEL, 2).astype(np.float32)
                 * (-math.log(10000.0) / D_MODEL))
    pe = np.zeros((N_SAMPLES, D_MODEL), dtype=np.float32)
    pe[:, 0::2] = np.sin(pos * div)
    pe[:, 1::2] = np.cos(pos * div)
    return np.tile(pe, (batch, 1))


def _build_rec(iota, mv, xb, yb, zb, wb):
    return jnp.where(iota == 0, mv,
           jnp.where(iota == 1, xb,
           jnp.where(iota == 2, yb,
           jnp.where(iota == 3, zb, wb))))


@functools.lru_cache(maxsize=None)
def _make_fps_kernel(batch, n_points, chunk):
    n_slices = chunk // (L * UNROLL)
    bpc = batch // NCORES

    mesh = plsc.VectorSubcoreMesh(core_axis_name="c", subcore_axis_name="s")

    @functools.partial(
        pl.kernel,
        out_type=jax.ShapeDtypeStruct((batch * N_SAMPLES * 4,), jnp.float32),
        mesh=mesh,
        compiler_params=pltpu.CompilerParams(needs_layout_passes=False),
        scratch_types=[
            pltpu.VMEM((chunk,), jnp.float32),
            pltpu.VMEM((chunk,), jnp.float32),
            pltpu.VMEM((chunk,), jnp.float32),
            pltpu.VMEM((chunk,), jnp.float32),
            pltpu.VMEM((chunk,), jnp.float32),
            pltpu.VMEM((L,), jnp.float32),
            pltpu.VMEM((WPB * L,), jnp.float32),
            pltpu.VMEM((N_SAMPLES * 4,), jnp.float32),
            pltpu.VMEM_SHARED((NSUB * L,), jnp.float32),
        ],
    )
    def fps_kernel(pts_hbm, out_hbm, x_ref, y_ref, z_ref, w_ref, dist_ref,
                   rec_ref, recs_ref, samp_ref, shared_ref):
        c = lax.axis_index("c")
        s = lax.axis_index("s")
        b = c * bpc + s // WPB
        g = s % WPB
        base = jnp.minimum(g * chunk, n_points - chunk) // 128 * 128
        iota = lax.iota(jnp.int32, L)
        gb = (s // WPB) * WPB
        in_group = iota < WPB
        ch_mask = iota < 4

        pltpu.sync_copy(pts_hbm.at[b, 0, pl.ds(base, chunk)], x_ref)
        pltpu.sync_copy(pts_hbm.at[b, 1, pl.ds(base, chunk)], y_ref)
        pltpu.sync_copy(pts_hbm.at[b, 2, pl.ds(base, chunk)], z_ref)
        pltpu.sync_copy(pts_hbm.at[b, 3, pl.ds(base, chunk)], w_ref)
        inf_v = jnp.full((L,), _INF)

        def init_step(t, _):
            dist_ref[pl.ds(t * L, L)] = inf_v
            return 0
        lax.fori_loop(0, chunk // L, init_step, 0, unroll=8)

        def publish_select(rec, i):
            rec_ref[...] = rec
            pltpu.sync_copy(rec_ref, shared_ref.at[pl.ds(s * L, L)])
            plsc.subcore_barrier()
            pltpu.sync_copy(shared_ref.at[pl.ds(gb * L, WPB * L)], recs_ref)
            plsc.subcore_barrier()
            m_all = plsc.load_gather(recs_ref, [iota * L], mask=in_group)
            mval = jnp.where(in_group, m_all, _NINF)
            mmax = jnp.max(mval)
            wg = jnp.min(jnp.where(mval == mmax, iota, jnp.int32(64)))
            wb = wg * L
            wbv = jnp.full((L,), wb, jnp.int32)
            cx = plsc.load_gather(recs_ref, [wbv + 1])
            cy = plsc.load_gather(recs_ref, [wbv + 2])
            cz = plsc.load_gather(recs_ref, [wbv + 3])

            @pl.when(g == 0)
            def _():
                vals = plsc.load_gather(recs_ref, [wbv + 1 + iota],
                                        mask=ch_mask)
                plsc.store_scatter(samp_ref, [i * 4 + iota], vals,
                                   mask=ch_mask)

            return cx, cy, cz

        def _lane_splat(v, k):
            return jnp.full((L,), jnp.sum(jnp.where(iota == k, v, 0.0)))

        x0 = _lane_splat(x_ref[pl.ds(0, L)], 0)
        y0 = _lane_splat(y_ref[pl.ds(0, L)], 0)
        z0 = _lane_splat(z_ref[pl.ds(0, L)], 0)
        w0 = _lane_splat(w_ref[pl.ds(0, L)], 0)
        m0 = jnp.where(g == 0, _INF, _NINF)
        rec0 = _build_rec(iota, jnp.full((L,), m0), x0, y0, z0, w0)
        carry0 = publish_select(rec0, jnp.int32(0))

        def fps_step(i, carry):
            cx, cy, cz = carry

            def upd(t, bc):
                bv, bp = bc
                off = t * L
                dx = x_ref[pl.ds(off, L)] - cx
                dy = y_ref[pl.ds(off, L)] - cy
                dz = z_ref[pl.ds(off, L)] - cz
                d2 = dx * dx + dy * dy + dz * dz
                nd = jnp.minimum(dist_ref[pl.ds(off, L)], d2)
                dist_ref[pl.ds(off, L)] = nd
                gt = nd > bv
                bv = jnp.where(gt, nd, bv)
                bp = jnp.where(gt, off + iota, bp)
                return bv, bp

            bv, bp = plsc.parallel_loop(
                0, chunk // L, unroll=UNROLL,
                carry=(jnp.full((L,), _NINF), iota))(upd)
            ml = jnp.max(bv)
            lp = jnp.min(jnp.where(bv == ml, bp, jnp.int32(1 << 30)))
            pv = jnp.full((L,), lp, jnp.int32)
            xb = plsc.load_gather(x_ref, [pv])
            yb = plsc.load_gather(y_ref, [pv])
            zb = plsc.load_gather(z_ref, [pv])
            wb = plsc.load_gather(w_ref, [pv])
            rec = _build_rec(iota, jnp.full((L,), ml), xb, yb, zb, wb)
            return publish_select(rec, i + 1)

        lax.fori_loop(0, N_SAMPLES - 1, fps_step, carry0)

        @pl.when(g == 0)
        def _():
            pltpu.sync_copy(samp_ref,
                            out_hbm.at[pl.ds(b * (N_SAMPLES * 4),
                                             N_SAMPLES * 4)])

    return fps_kernel


def _mlp_body(s_ref, w1_ref, b1_ref, w2_ref, b2_ref, w3_ref, b3_ref,
              pe_ref, o_ref):
    h = jnp.dot(s_ref[...], w1_ref[...],
                preferred_element_type=jnp.float32) + b1_ref[...]
    h = jnp.maximum(h, 0.0)
    h = jnp.dot(h, w2_ref[...],
                preferred_element_type=jnp.float32) + b2_ref[...]
    h = jnp.maximum(h, 0.0)
    o_ref[...] = (jnp.dot(h, w3_ref[...],
                          preferred_element_type=jnp.float32)
                  + b3_ref[...] + pe_ref[...])


def kernel(points, W1, b1, W2, b2, W3, b3):
    batch, n_points, _ = points.shape
    m = min(N_SAMPLES, n_points)
    gran = L * UNROLL
    chunk = ((n_points + WPB - 1) // WPB + gran - 1) // gran * gran

    pts_planes = jnp.transpose(points, (0, 2, 1))
    samp = _make_fps_kernel(batch, n_points, chunk)(pts_planes)
    s2d = samp.reshape(batch * N_SAMPLES, 4)

    pe_tiled = jnp.asarray(_make_pe_tiled(batch))
    out = pl.pallas_call(
        _mlp_body,
        out_shape=jax.ShapeDtypeStruct((batch * N_SAMPLES, D_MODEL),
                                       jnp.float32),
    )(s2d, W1.T, b1.reshape(1, -1), W2.T, b2.reshape(1, -1),
      W3.T, b3.reshape(1, -1), pe_tiled)
    return out.reshape(batch, m, D_MODEL)


# speedup vs baseline: 4.3338x; 1.0143x over previous
import functools
import math

import numpy as np
import jax
import jax.numpy as jnp
from jax import lax
from jax.experimental import pallas as pl
from jax.experimental.pallas import tpu as pltpu
from jax.experimental.pallas import tpu_sc as plsc

D_MODEL = 256
N_SAMPLES = 64
L = 16
NCORES = 2
NSUB = 16
WPB = 8
UNROLL = 8
_INF = np.float32(float("inf"))
_NINF = np.float32(-float("inf"))


def _make_pe_tiled(batch):
    pos = np.arange(N_SAMPLES)[:, None].astype(np.float32)
    div = np.exp(np.arange(0, D_MODEL, 2).astype(np.float32)
                 * (-math.log(10000.0) / D_MODEL))
    pe = np.zeros((N_SAMPLES, D_MODEL), dtype=np.float32)
    pe[:, 0::2] = np.sin(pos * div)
    pe[:, 1::2] = np.cos(pos * div)
    return np.tile(pe, (batch, 1))


def _build_rec(iota, mv, xb, yb, zb, wb):
    return jnp.where(iota == 0, mv,
           jnp.where(iota == 1, xb,
           jnp.where(iota == 2, yb,
           jnp.where(iota == 3, zb, wb))))


@functools.lru_cache(maxsize=None)
def _make_fps_kernel(batch, n_points, chunk):
    n_slices = chunk // (L * UNROLL)
    bpc = batch // NCORES

    mesh = plsc.VectorSubcoreMesh(core_axis_name="c", subcore_axis_name="s")

    @functools.partial(
        pl.kernel,
        out_type=jax.ShapeDtypeStruct((batch * N_SAMPLES * 4,), jnp.float32),
        mesh=mesh,
        compiler_params=pltpu.CompilerParams(needs_layout_passes=False),
        scratch_types=[
            pltpu.VMEM((chunk,), jnp.float32),
            pltpu.VMEM((chunk,), jnp.float32),
            pltpu.VMEM((chunk,), jnp.float32),
            pltpu.VMEM((chunk,), jnp.float32),
            pltpu.VMEM((chunk,), jnp.float32),
            pltpu.VMEM((L,), jnp.float32),
            pltpu.VMEM((WPB * L,), jnp.float32),
            pltpu.VMEM((N_SAMPLES * 4,), jnp.float32),
            pltpu.VMEM_SHARED((NSUB * L,), jnp.float32),
        ],
    )
    def fps_kernel(pts_hbm, out_hbm, x_ref, y_ref, z_ref, w_ref, dist_ref,
                   rec_ref, recs_ref, samp_ref, shared_ref):
        c = lax.axis_index("c")
        s = lax.axis_index("s")
        b = c * bpc + s // WPB
        g = s % WPB
        base = jnp.minimum(g * chunk, n_points - chunk) // 128 * 128
        iota = lax.iota(jnp.int32, L)
        gb = (s // WPB) * WPB
        in_group = iota < WPB
        ch_mask = iota < 4

        pltpu.sync_copy(pts_hbm.at[b, 0, pl.ds(base, chunk)], x_ref)
        pltpu.sync_copy(pts_hbm.at[b, 1, pl.ds(base, chunk)], y_ref)
        pltpu.sync_copy(pts_hbm.at[b, 2, pl.ds(base, chunk)], z_ref)
        pltpu.sync_copy(pts_hbm.at[b, 3, pl.ds(base, chunk)], w_ref)
        inf_v = jnp.full((L,), _INF)

        def init_step(t, _):
            dist_ref[pl.ds(t * L, L)] = inf_v
            return 0
        lax.fori_loop(0, chunk // L, init_step, 0, unroll=8)

        def publish_select(rec, i):
            rec_ref[...] = rec
            pltpu.sync_copy(rec_ref, shared_ref.at[pl.ds(s * L, L)])
            plsc.subcore_barrier()
            pltpu.sync_copy(shared_ref.at[pl.ds(gb * L, WPB * L)], recs_ref)
            m_all = plsc.load_gather(recs_ref, [iota * L], mask=in_group)
            mval = jnp.where(in_group, m_all, _NINF)
            mmax = jnp.max(mval)
            wg = jnp.min(jnp.where(mval == mmax, iota, jnp.int32(64)))
            wb = wg * L
            wbv = jnp.full((L,), wb, jnp.int32)
            cx = plsc.load_gather(recs_ref, [wbv + 1])
            cy = plsc.load_gather(recs_ref, [wbv + 2])
            cz = plsc.load_gather(recs_ref, [wbv + 3])

            @pl.when(g == 0)
            def _():
                vals = plsc.load_gather(recs_ref, [wbv + 1 + iota],
                                        mask=ch_mask)
                plsc.store_scatter(samp_ref, [i * 4 + iota], vals,
                                   mask=ch_mask)

            return cx, cy, cz

        def _lane_splat(v, k):
            return jnp.full((L,), jnp.sum(jnp.where(iota == k, v, 0.0)))

        x0 = _lane_splat(x_ref[pl.ds(0, L)], 0)
        y0 = _lane_splat(y_ref[pl.ds(0, L)], 0)
        z0 = _lane_splat(z_ref[pl.ds(0, L)], 0)
        w0 = _lane_splat(w_ref[pl.ds(0, L)], 0)
        m0 = jnp.where(g == 0, _INF, _NINF)
        rec0 = _build_rec(iota, jnp.full((L,), m0), x0, y0, z0, w0)
        carry0 = publish_select(rec0, jnp.int32(0))

        def fps_step(i, carry):
            cx, cy, cz = carry

            def upd(t, bc):
                bv, bp = bc
                off = t * L
                dx = x_ref[pl.ds(off, L)] - cx
                dy = y_ref[pl.ds(off, L)] - cy
                dz = z_ref[pl.ds(off, L)] - cz
                d2 = dx * dx + dy * dy + dz * dz
                nd = jnp.minimum(dist_ref[pl.ds(off, L)], d2)
                dist_ref[pl.ds(off, L)] = nd
                gt = nd > bv
                bv = jnp.where(gt, nd, bv)
                bp = jnp.where(gt, off + iota, bp)
                return bv, bp

            bv, bp = plsc.parallel_loop(
                0, chunk // L, unroll=UNROLL,
                carry=(jnp.full((L,), _NINF), iota))(upd)
            ml = jnp.max(bv)
            lp = jnp.min(jnp.where(bv == ml, bp, jnp.int32(1 << 30)))
            pv = jnp.full((L,), lp, jnp.int32)
            xb = plsc.load_gather(x_ref, [pv])
            yb = plsc.load_gather(y_ref, [pv])
            zb = plsc.load_gather(z_ref, [pv])
            wb = plsc.load_gather(w_ref, [pv])
            rec = _build_rec(iota, jnp.full((L,), ml), xb, yb, zb, wb)
            return publish_select(rec, i + 1)

        lax.fori_loop(0, N_SAMPLES - 1, fps_step, carry0)

        @pl.when(g == 0)
        def _():
            pltpu.sync_copy(samp_ref,
                            out_hbm.at[pl.ds(b * (N_SAMPLES * 4),
                                             N_SAMPLES * 4)])

    return fps_kernel


def _mlp_body(s_ref, w1_ref, b1_ref, w2_ref, b2_ref, w3_ref, b3_ref,
              pe_ref, o_ref):
    h = jnp.dot(s_ref[...], w1_ref[...],
                preferred_element_type=jnp.float32) + b1_ref[...]
    h = jnp.maximum(h, 0.0)
    h = jnp.dot(h, w2_ref[...],
                preferred_element_type=jnp.float32) + b2_ref[...]
    h = jnp.maximum(h, 0.0)
    o_ref[...] = (jnp.dot(h, w3_ref[...],
                          preferred_element_type=jnp.float32)
                  + b3_ref[...] + pe_ref[...])


def kernel(points, W1, b1, W2, b2, W3, b3):
    batch, n_points, _ = points.shape
    m = min(N_SAMPLES, n_points)
    gran = L * UNROLL
    chunk = ((n_points + WPB - 1) // WPB + gran - 1) // gran * gran

    pts_planes = jnp.transpose(points, (0, 2, 1))
    samp = _make_fps_kernel(batch, n_points, chunk)(pts_planes)
    s2d = samp.reshape(batch * N_SAMPLES, 4)

    pe_tiled = jnp.asarray(_make_pe_tiled(batch))
    out = pl.pallas_call(
        _mlp_body,
        out_shape=jax.ShapeDtypeStruct((batch * N_SAMPLES, D_MODEL),
                                       jnp.float32),
    )(s2d, W1.T, b1.reshape(1, -1), W2.T, b2.reshape(1, -1),
      W3.T, b3.reshape(1, -1), pe_tiled)
    return out.reshape(batch, m, D_MODEL)
